# Initial kernel scaffold; baseline (speedup 1.0000x reference)
import jax
import jax.numpy as jnp
from jax.experimental import pallas as pl


def kernel(x, tgt):
    raise NotImplementedError("write your pallas kernel here")


# speedup vs baseline: 7.8986x; 7.8986x over previous
import math

import jax
import jax.numpy as jnp
from jax.experimental import pallas as pl
from jax.experimental.pallas import tpu as pltpu

_SIZE = 32000
_PAD = 0
_SMOOTH = 0.1
_CONF = 1.0 - _SMOOTH
_EPS = _SMOOTH / (_SIZE - 2)
_C = _EPS * math.log(_EPS) * (_SIZE - 2) + _CONF * math.log(_CONF)

_ROWS_BLK = 128


def _ls_body(tgt_ref, x_ref, out_ref):
    r = pl.program_id(0)
    x = x_ref[...]
    tgt = tgt_ref[...]
    nonpad = tgt != _PAD
    col = jax.lax.broadcasted_iota(jnp.int32, x.shape, 1)
    w = jnp.where(col == tgt, _CONF, _EPS)
    w = jnp.where(col == _PAD, 0.0, w)
    w = jnp.where(nonpad, w, 0.0)
    cnt = jnp.sum(nonpad.astype(jnp.float32))
    part = _C * cnt - jnp.sum(w * x)

    @pl.when(r == 0)
    def _():
        out_ref[0, 0] = 0.0

    out_ref[0, 0] += part


def kernel(x, tgt):
    n = x.shape[0]
    tgt2 = tgt.astype(jnp.int32).reshape(n, 1)
    grid = (n // _ROWS_BLK,)
    out = pl.pallas_call(
        _ls_body,
        grid=grid,
        in_specs=[
            pl.BlockSpec((_ROWS_BLK, 1), lambda r: (r, 0)),
            pl.BlockSpec((_ROWS_BLK, _SIZE), lambda r: (r, 0)),
        ],
        out_specs=pl.BlockSpec(memory_space=pltpu.SMEM),
        out_shape=jax.ShapeDtypeStruct((1, 1), jnp.float32),
    )(tgt2, x)
    return out[0, 0]
